# baseline (speedup 1.0000x reference)
import functools

import jax
import jax.numpy as jnp
from jax.experimental import pallas as pl


def _conv_body(TH, W, WP, x1_ref, x2_ref, m1_ref, m2_ref, w_ref, b_ref,
               out_ref):
    C = x1_ref.shape[1]
    NA = TH * WP
    xf = jnp.concatenate([x1_ref[0], x2_ref[0][:, : 2 * WP]], axis=1)
    xsh = jnp.concatenate(
        [xf, jnp.roll(xf, -1, axis=1), jnp.roll(xf, -2, axis=1)], axis=0
    )
    acc = None
    for dy in range(3):
        wdy = w_ref[dy * 3 * C : (dy + 1) * 3 * C, :]
        xsl = xsh[:, dy * WP : dy * WP + NA]
        part = jax.lax.dot_general(
            wdy, xsl,
            dimension_numbers=(((0,), (0,)), ((), ())),
            preferred_element_type=jnp.float32,
        )
        acc = part if acc is None else acc + part
    Cout = acc.shape[0]
    acc = acc + b_ref[...]

    mf = jnp.concatenate([m1_ref[0], m2_ref[0][:, : 2 * WP]], axis=1)
    msh = mf + jnp.roll(mf, -1, axis=1) + jnp.roll(mf, -2, axis=1)
    cover = (msh[:, 0:NA] + msh[:, WP : WP + NA]
             + msh[:, 2 * WP : 2 * WP + NA])
    res = jnp.where(cover > 0.0, acc, 0.0)
    out_ref[0] = res.reshape(Cout, TH, WP)[:, :, :W]


def kernel(x, mask, weight, bias):
    B, C, H, W = x.shape
    Cout, _, KH, KW = weight.shape
    TH = 32
    WP = 256
    ntiles = H // TH
    HP = H + 16

    xp = jnp.pad(x.astype(jnp.bfloat16),
                 ((0, 0), (0, 0), (1, HP - H - 1), (1, WP - W - 1)))
    xpf = xp.reshape(B, C, HP * WP)
    mp = jnp.pad(mask, ((0, 0), (0, 0), (1, HP - H - 1), (1, WP - W - 1)))
    mpf = mp.reshape(B, 1, HP * WP)

    wfull = weight.transpose(2, 3, 1, 0).reshape(KH * KW * C, Cout)
    wfull = wfull.astype(jnp.bfloat16)
    b2 = bias.reshape(Cout, 1)

    r = TH // 8
    grid = (B, ntiles)
    out = pl.pallas_call(
        functools.partial(_conv_body, TH, W, WP),
        grid=grid,
        in_specs=[
            pl.BlockSpec((1, C, TH * WP), lambda b, i: (b, 0, i)),
            pl.BlockSpec((1, C, 8 * WP), lambda b, i: (b, 0, r * i + r)),
            pl.BlockSpec((1, 1, TH * WP), lambda b, i: (b, 0, i)),
            pl.BlockSpec((1, 1, 8 * WP), lambda b, i: (b, 0, r * i + r)),
            pl.BlockSpec((KH * KW * C, Cout), lambda b, i: (0, 0)),
            pl.BlockSpec((Cout, 1), lambda b, i: (0, 0)),
        ],
        out_specs=pl.BlockSpec((1, Cout, TH, W), lambda b, i: (b, 0, i, 0)),
        out_shape=jax.ShapeDtypeStruct((B, Cout, H, W), jnp.float32),
    )(xpf, xpf, mpf, mpf, wfull, b2)
    return out


# speedup vs baseline: 1.7489x; 1.7489x over previous
import functools

import jax
import jax.numpy as jnp
from jax.experimental import pallas as pl
from jax.experimental.pallas import tpu as pltpu


def _conv_body(TH, W, WP, x1_ref, xn_ref, m1_ref, mn_ref, w_ref, b_ref,
               out_ref, xtop_ref, mtop_ref):
    i = pl.program_id(1)
    ntiles = pl.num_programs(1)
    C = x1_ref.shape[1]
    PADR = WP - W - 1

    xa = jnp.pad(x1_ref[0].astype(jnp.bfloat16),
                 ((0, 0), (0, 0), (1, PADR)))
    xn = jnp.pad(xn_ref[0, :, 0:1, :].astype(jnp.bfloat16),
                 ((0, 0), (0, 0), (1, PADR)))
    xn = jnp.where(i == ntiles - 1, jnp.zeros_like(xn), xn)
    xt = jnp.where(i == 0, jnp.zeros_like(xtop_ref), xtop_ref[...])
    xblk = jnp.concatenate([xt, xa, xn], axis=1)
    xtop_ref[...] = xa[:, TH - 1 : TH, :]

    xfl = xblk.reshape(C, (TH + 2) * WP)
    xsh = jnp.concatenate(
        [xfl, jnp.roll(xfl, -1, axis=1), jnp.roll(xfl, -2, axis=1)], axis=0
    )
    acc = None
    for dy in range(3):
        wdy = w_ref[dy * 3 * C : (dy + 1) * 3 * C, :]
        xsl = xsh[:, dy * WP : dy * WP + TH * WP]
        part = jax.lax.dot_general(
            wdy, xsl,
            dimension_numbers=(((0,), (0,)), ((), ())),
            preferred_element_type=jnp.float32,
        )
        acc = part if acc is None else acc + part
    Cout = acc.shape[0]
    acc = (acc + b_ref[...]).reshape(Cout, TH, WP)

    ma = jnp.pad(m1_ref[0, 0], ((0, 0), (1, PADR)))
    mn = jnp.pad(mn_ref[0, 0, 0:1, :], ((0, 0), (1, PADR)))
    mn = jnp.where(i == ntiles - 1, jnp.zeros_like(mn), mn)
    mt = jnp.where(i == 0, jnp.zeros_like(mtop_ref), mtop_ref[...])
    mblk = jnp.concatenate([mt[0:1], ma, mn], axis=0)
    mtop_ref[...] = ma[TH - 1 : TH, :]
    msh = mblk + jnp.roll(mblk, -1, axis=-1) + jnp.roll(mblk, -2, axis=-1)
    cover = msh[0:TH] + msh[1 : TH + 1] + msh[2 : TH + 2]
    active = cover > 0.0
    res = jnp.where(active[None, :, :], acc, 0.0)
    out_ref[0] = res[:, :, :W]


def kernel(x, mask, weight, bias):
    B, C, H, W = x.shape
    Cout, _, KH, KW = weight.shape
    TH = 32
    TB = 8
    WP = 256
    ntiles = H // TH
    nlast = H // TB - 1

    wfull = weight.transpose(2, 3, 1, 0).reshape(KH * KW * C, Cout)
    wfull = wfull.astype(jnp.bfloat16)
    b2 = bias.reshape(Cout, 1)

    r = TH // TB
    grid = (B, ntiles)
    out = pl.pallas_call(
        functools.partial(_conv_body, TH, W, WP),
        grid=grid,
        in_specs=[
            pl.BlockSpec((1, C, TH, W), lambda b, i: (b, 0, i, 0)),
            pl.BlockSpec((1, C, TB, W),
                         lambda b, i: (b, 0, jnp.minimum(r * i + r, nlast), 0)),
            pl.BlockSpec((1, 1, TH, W), lambda b, i: (b, 0, i, 0)),
            pl.BlockSpec((1, 1, TB, W),
                         lambda b, i: (b, 0, jnp.minimum(r * i + r, nlast), 0)),
            pl.BlockSpec((KH * KW * C, Cout), lambda b, i: (0, 0)),
            pl.BlockSpec((Cout, 1), lambda b, i: (0, 0)),
        ],
        out_specs=pl.BlockSpec((1, Cout, TH, W), lambda b, i: (b, 0, i, 0)),
        out_shape=jax.ShapeDtypeStruct((B, Cout, H, W), jnp.float32),
        scratch_shapes=[
            pltpu.VMEM((C, 1, WP), jnp.bfloat16),
            pltpu.VMEM((1, WP), jnp.float32),
        ],
    )(x, x, mask, mask, wfull, b2)
    return out


# speedup vs baseline: 2.4440x; 1.3974x over previous
import functools

import jax
import jax.numpy as jnp
from jax.experimental import pallas as pl
from jax.experimental.pallas import tpu as pltpu


def _conv_body(TH, W, WP, x1_ref, xn_ref, m1_ref, mn_ref, w_ref, b_ref,
               out_ref, xtop_ref, mtop_ref):
    i = pl.program_id(1)
    ntiles = pl.num_programs(1)
    C = x1_ref.shape[1]
    PADR = WP - W - 1

    xa = jnp.pad(x1_ref[0].astype(jnp.bfloat16),
                 ((0, 0), (0, 0), (1, PADR)))
    xn = jnp.pad(xn_ref[0, :, 0:1, :].astype(jnp.bfloat16),
                 ((0, 0), (0, 0), (1, PADR)))
    xn = jnp.where(i == ntiles - 1, jnp.zeros_like(xn), xn)
    xt = jnp.where(i == 0, jnp.zeros_like(xtop_ref), xtop_ref[...])
    xblk = jnp.concatenate([xt, xa, xn], axis=1)
    xtop_ref[...] = xa[:, TH - 1 : TH, :]

    xfl = xblk.reshape(C, (TH + 2) * WP)
    xsh = jnp.concatenate(
        [xfl, jnp.roll(xfl, -1, axis=1), jnp.roll(xfl, -2, axis=1)], axis=0
    )
    N = TH * WP
    xcat = jnp.concatenate(
        [xsh[:, 0:N], xsh[:, WP : WP + N], xsh[:, 2 * WP : 2 * WP + N]],
        axis=0,
    )
    acc = jax.lax.dot_general(
        w_ref[...], xcat,
        dimension_numbers=(((0,), (0,)), ((), ())),
        preferred_element_type=jnp.float32,
    )
    Cout = acc.shape[0]
    acc = (acc + b_ref[...]).reshape(Cout, TH, WP)

    ma = jnp.pad(m1_ref[0, 0], ((0, 0), (1, PADR)))
    mn = jnp.pad(mn_ref[0, 0, 0:1, :], ((0, 0), (1, PADR)))
    mn = jnp.where(i == ntiles - 1, jnp.zeros_like(mn), mn)
    mt = jnp.where(i == 0, jnp.zeros_like(mtop_ref), mtop_ref[...])
    mblk = jnp.concatenate([mt[0:1], ma, mn], axis=0)
    mtop_ref[...] = ma[TH - 1 : TH, :]
    msh = mblk + jnp.roll(mblk, -1, axis=-1) + jnp.roll(mblk, -2, axis=-1)
    cover = msh[0:TH] + msh[1 : TH + 1] + msh[2 : TH + 2]
    active = cover > 0.0
    res = jnp.where(active[None, :, :], acc, 0.0)
    out_ref[0] = res[:, :, :W]


def kernel(x, mask, weight, bias):
    B, C, H, W = x.shape
    Cout, _, KH, KW = weight.shape
    TH = 32
    TB = 8
    WP = 256
    ntiles = H // TH
    nlast = H // TB - 1

    wfull = weight.transpose(2, 3, 1, 0).reshape(KH * KW * C, Cout)
    wfull = wfull.astype(jnp.bfloat16)
    b2 = bias.reshape(Cout, 1)

    r = TH // TB
    grid = (B, ntiles)
    out = pl.pallas_call(
        functools.partial(_conv_body, TH, W, WP),
        grid=grid,
        in_specs=[
            pl.BlockSpec((1, C, TH, W), lambda b, i: (b, 0, i, 0)),
            pl.BlockSpec((1, C, TB, W),
                         lambda b, i: (b, 0, jnp.minimum(r * i + r, nlast), 0)),
            pl.BlockSpec((1, 1, TH, W), lambda b, i: (b, 0, i, 0)),
            pl.BlockSpec((1, 1, TB, W),
                         lambda b, i: (b, 0, jnp.minimum(r * i + r, nlast), 0)),
            pl.BlockSpec((KH * KW * C, Cout), lambda b, i: (0, 0)),
            pl.BlockSpec((Cout, 1), lambda b, i: (0, 0)),
        ],
        out_specs=pl.BlockSpec((1, Cout, TH, W), lambda b, i: (b, 0, i, 0)),
        out_shape=jax.ShapeDtypeStruct((B, Cout, H, W), jnp.float32),
        scratch_shapes=[
            pltpu.VMEM((C, 1, WP), jnp.bfloat16),
            pltpu.VMEM((1, WP), jnp.float32),
        ],
    )(x, x, mask, mask, wfull, b2)
    return out


# speedup vs baseline: 2.5397x; 1.0392x over previous
import functools

import jax
import jax.numpy as jnp
from jax.experimental import pallas as pl
from jax.experimental.pallas import tpu as pltpu


def _conv_body(TH, W, WP, x1_ref, xn_ref, m1_ref, mn_ref, w_ref, b_ref,
               out_ref, xtop_ref, mtop_ref):
    i = pl.program_id(1)
    ntiles = pl.num_programs(1)
    C = x1_ref.shape[1]
    PADR = WP - W - 1

    xa = jnp.pad(x1_ref[0].astype(jnp.bfloat16),
                 ((0, 0), (0, 0), (1, PADR)))
    xn = jnp.pad(xn_ref[0, :, 0:1, :].astype(jnp.bfloat16),
                 ((0, 0), (0, 0), (1, PADR)))
    xn = jnp.where(i == ntiles - 1, jnp.zeros_like(xn), xn)
    xt = jnp.where(i == 0, jnp.zeros_like(xtop_ref), xtop_ref[...])
    xblk = jnp.concatenate([xt, xa, xn], axis=1)
    xtop_ref[...] = xa[:, TH - 1 : TH, :]

    xfl = xblk.reshape(C, (TH + 2) * WP)
    xsh = jnp.concatenate(
        [xfl, jnp.roll(xfl, -1, axis=1), jnp.roll(xfl, -2, axis=1)], axis=0
    )
    N = TH * WP
    xcat = jnp.concatenate(
        [xsh[:, 0:N], xsh[:, WP : WP + N], xsh[:, 2 * WP : 2 * WP + N]],
        axis=0,
    )
    acc = jax.lax.dot_general(
        w_ref[...], xcat,
        dimension_numbers=(((0,), (0,)), ((), ())),
        preferred_element_type=jnp.float32,
    )
    Cout = acc.shape[0]
    acc = (acc + b_ref[...]).reshape(Cout, TH, WP)

    ma = jnp.pad(m1_ref[0, 0], ((0, 0), (1, PADR)))
    mn = jnp.pad(mn_ref[0, 0, 0:1, :], ((0, 0), (1, PADR)))
    mn = jnp.where(i == ntiles - 1, jnp.zeros_like(mn), mn)
    mt = jnp.where(i == 0, jnp.zeros_like(mtop_ref), mtop_ref[...])
    mblk = jnp.concatenate([mt[0:1], ma, mn], axis=0)
    mtop_ref[...] = ma[TH - 1 : TH, :]
    msh = mblk + jnp.roll(mblk, -1, axis=-1) + jnp.roll(mblk, -2, axis=-1)
    cover = msh[0:TH] + msh[1 : TH + 1] + msh[2 : TH + 2]
    active = cover > 0.0
    res = jnp.where(active[None, :, :], acc, 0.0)
    out_ref[0] = res[:, :, :W]


def kernel(x, mask, weight, bias):
    B, C, H, W = x.shape
    Cout, _, KH, KW = weight.shape
    TH = 56
    TB = 8
    WP = 256
    ntiles = H // TH
    nlast = H // TB - 1

    wfull = weight.transpose(2, 3, 1, 0).reshape(KH * KW * C, Cout)
    wfull = wfull.astype(jnp.bfloat16)
    b2 = bias.reshape(Cout, 1)

    r = TH // TB
    grid = (B, ntiles)
    out = pl.pallas_call(
        functools.partial(_conv_body, TH, W, WP),
        grid=grid,
        in_specs=[
            pl.BlockSpec((1, C, TH, W), lambda b, i: (b, 0, i, 0)),
            pl.BlockSpec((1, C, TB, W),
                         lambda b, i: (b, 0, jnp.minimum(r * i + r, nlast), 0)),
            pl.BlockSpec((1, 1, TH, W), lambda b, i: (b, 0, i, 0)),
            pl.BlockSpec((1, 1, TB, W),
                         lambda b, i: (b, 0, jnp.minimum(r * i + r, nlast), 0)),
            pl.BlockSpec((KH * KW * C, Cout), lambda b, i: (0, 0)),
            pl.BlockSpec((Cout, 1), lambda b, i: (0, 0)),
        ],
        out_specs=pl.BlockSpec((1, Cout, TH, W), lambda b, i: (b, 0, i, 0)),
        out_shape=jax.ShapeDtypeStruct((B, Cout, H, W), jnp.float32),
        scratch_shapes=[
            pltpu.VMEM((C, 1, WP), jnp.bfloat16),
            pltpu.VMEM((1, WP), jnp.float32),
        ],
    )(x, x, mask, mask, wfull, b2)
    return out
